# baseline (speedup 1.0000x reference)
import functools

import jax
import jax.numpy as jnp
from jax.experimental import pallas as pl

_NEIGHBOR_N = 5


def _fused_kernel(eb_ref, e_ref, lb_ref, la_ref, out_ref, *, k):
    E = e_ref[:]
    eb = eb_ref[:]
    lb = lb_ref[0, :]
    la = la_ref[0, :]
    fenzi = jax.lax.dot_general(eb, E, (((1,), (1,)), ((), ())),
                                preferred_element_type=jnp.float32)
    sim = fenzi / (lb[:, None] * la[None, :])
    m = jnp.max(sim, axis=1)
    psum = jnp.sum(jnp.exp((sim - m[:, None]).astype(jnp.bfloat16))
                   .astype(jnp.float32), axis=1)
    iota = jax.lax.broadcasted_iota(jnp.int32, sim.shape, 1)
    simw = sim
    vals, idxs = [], []
    for _ in range(k):
        v = jnp.max(simw, axis=1)
        idx = jnp.min(jnp.where(simw == v[:, None], iota, jnp.int32(2**30)),
                      axis=1)
        vals.append(v)
        idxs.append(idx)
        simw = jnp.where(iota == idx[:, None], -jnp.inf, simw)
    p = [jnp.exp(v - m) / psum for v in vals]
    ex = [jnp.exp(pj - p[0]) for pj in p]
    denom = ex[0]
    for e in ex[1:]:
        denom = denom + e
    W = jnp.where(iota == idxs[0][:, None], (ex[0] / denom)[:, None], 0.0)
    for j in range(1, k):
        W = W + jnp.where(iota == idxs[j][:, None], (ex[j] / denom)[:, None],
                          0.0)
    out_ref[:] = jax.lax.dot_general(W, E, (((1,), (0,)), ((), ())),
                                     preferred_element_type=jnp.float32,
                                     precision=jax.lax.Precision.HIGHEST)


@jax.jit
def kernel(sess_emb):
    B, H = sess_emb.shape
    k = min(_NEIGHBOR_N, B)
    R = 256 if B % 256 == 0 else B
    fenmu_l = jnp.sqrt(jnp.sum(sess_emb * sess_emb + 1e-06, axis=1))[None, :]
    return pl.pallas_call(
        functools.partial(_fused_kernel, k=k),
        grid=(B // R,),
        in_specs=[
            pl.BlockSpec((R, H), lambda i: (i, 0)),
            pl.BlockSpec((B, H), lambda i: (0, 0)),
            pl.BlockSpec((1, R), lambda i: (0, i)),
            pl.BlockSpec((1, B), lambda i: (0, 0)),
        ],
        out_specs=pl.BlockSpec((R, H), lambda i: (i, 0)),
        out_shape=jax.ShapeDtypeStruct((B, H), jnp.float32),
    )(sess_emb, sess_emb, fenmu_l, fenmu_l)


# speedup vs baseline: 1.7538x; 1.7538x over previous
import functools

import jax
import jax.numpy as jnp
from jax.experimental import pallas as pl

_NEIGHBOR_N = 5


def _fused_kernel(eb_ref, e_ref, lb_ref, la_ref, out_ref, *, k):
    E = e_ref[:]
    eb = eb_ref[:]
    lb = lb_ref[0, :]
    la = la_ref[0, :]
    fenzi = jax.lax.dot_general(eb, E, (((1,), (1,)), ((), ())),
                                preferred_element_type=jnp.float32)
    sim = fenzi / (lb[:, None] * la[None, :])
    simw = sim
    vals = []
    for _ in range(k):
        v = jnp.max(simw, axis=1)
        vals.append(v)
        simw = jnp.where(simw == v[:, None], -jnp.inf, simw)
    m = vals[0]
    psum = jnp.sum(jnp.exp(sim - m[:, None]), axis=1)
    p = [jnp.exp(v - m) / psum for v in vals]
    ex = [jnp.exp(pj - p[0]) for pj in p]
    denom = ex[0]
    for e in ex[1:]:
        denom = denom + e
    W = jnp.zeros_like(sim)
    for j in range(k - 1, -1, -1):
        W = jnp.where(sim == vals[j][:, None], (ex[j] / denom)[:, None], W)
    out_ref[:] = jax.lax.dot_general(W, E, (((1,), (0,)), ((), ())),
                                     preferred_element_type=jnp.float32,
                                     precision=jax.lax.Precision.HIGHEST)


@jax.jit
def kernel(sess_emb):
    B, H = sess_emb.shape
    k = min(_NEIGHBOR_N, B)
    R = 256 if B % 256 == 0 else B
    fenmu_l = jnp.sqrt(jnp.sum(sess_emb * sess_emb + 1e-06, axis=1))[None, :]
    return pl.pallas_call(
        functools.partial(_fused_kernel, k=k),
        grid=(B // R,),
        in_specs=[
            pl.BlockSpec((R, H), lambda i: (i, 0)),
            pl.BlockSpec((B, H), lambda i: (0, 0)),
            pl.BlockSpec((1, R), lambda i: (0, i)),
            pl.BlockSpec((1, B), lambda i: (0, 0)),
        ],
        out_specs=pl.BlockSpec((R, H), lambda i: (i, 0)),
        out_shape=jax.ShapeDtypeStruct((B, H), jnp.float32),
    )(sess_emb, sess_emb, fenmu_l, fenmu_l)


# speedup vs baseline: 1.8115x; 1.0329x over previous
import functools

import jax
import jax.numpy as jnp
from jax.experimental import pallas as pl

_NEIGHBOR_N = 5


def _fused_kernel(eb_ref, e_ref, lb_ref, la_ref, out_ref, *, k):
    E = e_ref[:]
    eb = eb_ref[:]
    lb = lb_ref[0, :]
    la = la_ref[0, :]
    fenzi = jax.lax.dot_general(eb, E, (((1,), (1,)), ((), ())),
                                preferred_element_type=jnp.float32)
    sim = fenzi / (lb[:, None] * la[None, :])
    simw = sim
    vals = []
    for _ in range(k):
        v = jnp.max(simw, axis=1)
        vals.append(v)
        simw = jnp.where(simw == v[:, None], -jnp.inf, simw)
    m = vals[0]
    expsim = jnp.exp(sim - m[:, None])
    psum = jnp.sum(expsim, axis=1)
    p = [jnp.exp(v - m) / psum for v in vals]
    ex = [jnp.exp(pj - p[0]) for pj in p]
    denom = ex[0]
    for e in ex[1:]:
        denom = denom + e
    u = jnp.exp(expsim / psum[:, None] - p[0][:, None])
    W = jnp.where(sim >= vals[k - 1][:, None], u, 0.0)
    out = jax.lax.dot_general(W, E, (((1,), (0,)), ((), ())),
                              preferred_element_type=jnp.float32,
                              precision=jax.lax.Precision.HIGHEST)
    out_ref[:] = out / denom[:, None]


@jax.jit
def kernel(sess_emb):
    B, H = sess_emb.shape
    k = min(_NEIGHBOR_N, B)
    R = 256 if B % 256 == 0 else B
    fenmu_l = jnp.sqrt(jnp.sum(sess_emb * sess_emb + 1e-06, axis=1))[None, :]
    return pl.pallas_call(
        functools.partial(_fused_kernel, k=k),
        grid=(B // R,),
        in_specs=[
            pl.BlockSpec((R, H), lambda i: (i, 0)),
            pl.BlockSpec((B, H), lambda i: (0, 0)),
            pl.BlockSpec((1, R), lambda i: (0, i)),
            pl.BlockSpec((1, B), lambda i: (0, 0)),
        ],
        out_specs=pl.BlockSpec((R, H), lambda i: (i, 0)),
        out_shape=jax.ShapeDtypeStruct((B, H), jnp.float32),
    )(sess_emb, sess_emb, fenmu_l, fenmu_l)


# speedup vs baseline: 2.9159x; 1.6097x over previous
import functools

import jax
import jax.numpy as jnp
from jax.experimental import pallas as pl

_NEIGHBOR_N = 5


def _fused_kernel(eb_ref, e_ref, lb_ref, la_ref, out_ref, *, k):
    E = e_ref[:]
    eb = eb_ref[:]
    lb = lb_ref[0, :]
    la = la_ref[0, :]
    fenzi = jax.lax.dot_general(eb, E, (((1,), (1,)), ((), ())),
                                preferred_element_type=jnp.float32)
    sim = fenzi / (lb[:, None] * la[None, :])
    vals = [jnp.max(sim, axis=1)]
    for _ in range(k - 1):
        v = jnp.max(jnp.where(sim < vals[-1][:, None], sim, -jnp.inf), axis=1)
        vals.append(v)
    m = vals[0]
    psum = jnp.sum(jnp.exp(sim - m[:, None]), axis=1)
    p = [jnp.exp(v - m) / psum for v in vals]
    ex = [jnp.exp(pj - p[0]) for pj in p]
    denom = ex[0]
    for e in ex[1:]:
        denom = denom + e
    u = jnp.exp(jnp.exp(sim - m[:, None]) / psum[:, None] - p[0][:, None])
    W = jnp.where(sim >= vals[k - 1][:, None], u, 0.0)
    out = jax.lax.dot_general(W, E, (((1,), (0,)), ((), ())),
                              preferred_element_type=jnp.float32)
    out_ref[:] = out / denom[:, None]


@jax.jit
def kernel(sess_emb):
    B, H = sess_emb.shape
    k = min(_NEIGHBOR_N, B)
    R = 256 if B % 256 == 0 else B
    fenmu_l = jnp.sqrt(jnp.sum(sess_emb * sess_emb + 1e-06, axis=1))[None, :]
    return pl.pallas_call(
        functools.partial(_fused_kernel, k=k),
        grid=(B // R,),
        in_specs=[
            pl.BlockSpec((R, H), lambda i: (i, 0)),
            pl.BlockSpec((B, H), lambda i: (0, 0)),
            pl.BlockSpec((1, R), lambda i: (0, i)),
            pl.BlockSpec((1, B), lambda i: (0, 0)),
        ],
        out_specs=pl.BlockSpec((R, H), lambda i: (i, 0)),
        out_shape=jax.ShapeDtypeStruct((B, H), jnp.float32),
    )(sess_emb, sess_emb, fenmu_l, fenmu_l)
